# baseline (speedup 1.0000x reference)
import jax
import jax.numpy as jnp
from jax.experimental import pallas as pl
from jax.experimental.pallas import tpu as pltpu

HIDDEN = 1024
DECISIONS = 1024
ENV_DEPTH = 8
BATCH = 16
K = 64


def _softmax_cand(logits, temp, scores_col):
    x = logits / temp
    m = jnp.max(x, axis=-1, keepdims=True)
    shifted = x - m
    lse = jnp.log(jnp.sum(jnp.exp(shifted), axis=-1, keepdims=True))
    return scores_col + (shifted - lse)


def _mono_body(root_ref, emb_ref, wih_ref, whh_ref, bih_ref, bhh_ref,
               wout_ref, bout_ref, t_ref, oseq_ref, osc_ref, workT_ref):
    t = t_ref[0]
    D = DECISIONS
    H = HIDDEN
    dn = (((1,), (1,)), ((), ()))
    lane = jax.lax.broadcasted_iota(jnp.int32, (1, K), 1)
    rowi = jax.lax.broadcasted_iota(jnp.int32, (K, 1), 0)
    colD = jax.lax.broadcasted_iota(jnp.int32, (K, D), 1)
    colK = jax.lax.broadcasted_iota(jnp.int32, (K, K), 1)
    colS = jax.lax.broadcasted_iota(jnp.int32, (K, ENV_DEPTH), 1)
    big = jnp.int32(2 ** 31 - 1)
    neg = jnp.float32(-jnp.inf)

    def top64(cand):
        nb = cand.shape[0]
        rowb = jax.lax.broadcasted_iota(jnp.int32, (nb, D), 0)
        col1 = jax.lax.broadcasted_iota(jnp.int32, (1, D), 1)
        lane64 = jax.lax.broadcasted_iota(jnp.int32, (1, K), 1)
        if nb < K:
            workT_ref[...] = jnp.full((D, K), neg, jnp.float32)
        workT_ref[:, 0:nb] = jnp.transpose(cand)
        colmax0 = jnp.max(cand, axis=0, keepdims=True)
        colarg0 = jnp.min(jnp.where(cand == colmax0, rowb, big),
                          axis=0, keepdims=True)

        def body(j, carry):
            colmax, colarg, vals_lane, vals_col, a_col, bi_col = carry
            m = jnp.max(colmax)
            rmin = jnp.min(jnp.where(colmax == m, colarg, big))
            c = jnp.min(jnp.where((colmax == m) & (colarg == rmin), col1, big))
            colv = workT_ref[pl.ds(c, 1), :]
            new_colv = jnp.where(lane64 == rmin, neg, colv)
            workT_ref[pl.ds(c, 1), :] = new_colv
            ncm = jnp.max(new_colv)
            nar = jnp.min(jnp.where(new_colv == ncm, lane64, big))
            hit = col1 == c
            colmax = jnp.where(hit, ncm, colmax)
            colarg = jnp.where(hit, nar, colarg)
            sel = rowi == j
            vals_lane = jnp.where(lane == j, m, vals_lane)
            vals_col = jnp.where(sel, m, vals_col)
            a_col = jnp.where(sel, c, a_col)
            bi_col = jnp.where(sel, rmin, bi_col)
            return colmax, colarg, vals_lane, vals_col, a_col, bi_col

        init = (colmax0, colarg0,
                jnp.zeros((1, K), jnp.float32), jnp.zeros((K, 1), jnp.float32),
                jnp.zeros((K, 1), jnp.int32), jnp.zeros((K, 1), jnp.int32))
        _, _, vals_lane, vals_col, a_col, bi_col = jax.lax.fori_loop(
            0, K, body, init)
        return vals_lane, vals_col, a_col, bi_col

    states = root_ref[...]
    scores_col = jnp.zeros((1, 1), jnp.float32)
    seqs_f = jnp.zeros((K, ENV_DEPTH), jnp.float32)
    vals_lane = None
    for d in range(ENV_DEPTH):
        nb = states.shape[0]
        logits = jnp.dot(states, wout_ref[...],
                         preferred_element_type=jnp.float32) + bout_ref[...]
        cand = _softmax_cand(logits, t, scores_col)
        vals_lane, scores_col, a_col, bi_col = top64(cand)
        hp = jax.lax.Precision.HIGHEST
        onehot_a = (colD == a_col).astype(jnp.float32)
        x = jnp.dot(onehot_a, emb_ref[...], precision=hp,
                    preferred_element_type=jnp.float32)
        if d == 0:
            h = jnp.broadcast_to(states, (K, H))
            seqs_f = jnp.where(colS == d, a_col.astype(jnp.float32), seqs_f)
        else:
            onehot_b = (colK == bi_col).astype(jnp.float32)
            h = jnp.dot(onehot_b, states, precision=hp,
                        preferred_element_type=jnp.float32)
            seqs_f = jnp.dot(onehot_b, seqs_f, precision=hp,
                             preferred_element_type=jnp.float32)
            seqs_f = jnp.where(colS == d, a_col.astype(jnp.float32), seqs_f)
        gi = jax.lax.dot_general(x, wih_ref[...], dn,
                                 preferred_element_type=jnp.float32) + bih_ref[...]
        gh = jax.lax.dot_general(h, whh_ref[...], dn,
                                 preferred_element_type=jnp.float32) + bhh_ref[...]
        i_r, i_z, i_n = gi[:, 0:H], gi[:, H:2 * H], gi[:, 2 * H:3 * H]
        h_r, h_z, h_n = gh[:, 0:H], gh[:, H:2 * H], gh[:, 2 * H:3 * H]
        r = jax.nn.sigmoid(i_r + h_r)
        z = jax.nn.sigmoid(i_z + h_z)
        n = jnp.tanh(i_n + r * h_n)
        states = (1.0 - z) * n + z * h

    osc_ref[...] = jnp.broadcast_to(vals_lane, (BATCH, K))
    seqs_i = seqs_f.astype(jnp.int32)
    for b in range(BATCH):
        oseq_ref[b] = seqs_i


def kernel(env, root_state, emb, W_ih, W_hh, b_ih, b_hh, W_out, b_out, beams, temp):
    H = root_state.shape[1]
    D = W_out.shape[1]
    tempf = jnp.asarray(temp, jnp.float32).reshape(1)
    out_seqs, out_scores = pl.pallas_call(
        _mono_body,
        in_specs=[
            pl.BlockSpec(memory_space=pltpu.VMEM),
            pl.BlockSpec(memory_space=pltpu.VMEM),
            pl.BlockSpec(memory_space=pltpu.VMEM),
            pl.BlockSpec(memory_space=pltpu.VMEM),
            pl.BlockSpec(memory_space=pltpu.VMEM),
            pl.BlockSpec(memory_space=pltpu.VMEM),
            pl.BlockSpec(memory_space=pltpu.VMEM),
            pl.BlockSpec(memory_space=pltpu.VMEM),
            pl.BlockSpec(memory_space=pltpu.SMEM),
        ],
        out_shape=(
            jax.ShapeDtypeStruct((BATCH, K, ENV_DEPTH), jnp.int32),
            jax.ShapeDtypeStruct((BATCH, K), jnp.float32),
        ),
        scratch_shapes=[pltpu.VMEM((DECISIONS, K), jnp.float32)],
    )(root_state, emb, W_ih, W_hh,
      b_ih.reshape(1, 3 * H), b_hh.reshape(1, 3 * H),
      W_out, b_out.reshape(1, D), tempf)
    return (out_seqs, out_scores)


# speedup vs baseline: 2.0695x; 2.0695x over previous
import jax
import jax.numpy as jnp
from jax.experimental import pallas as pl
from jax.experimental.pallas import tpu as pltpu

HIDDEN = 1024
DECISIONS = 1024
ENV_DEPTH = 8
BATCH = 16
K = 64


def _softmax_cand(logits, temp, scores_col):
    x = logits / temp
    m = jnp.max(x, axis=-1, keepdims=True)
    shifted = x - m
    lse = jnp.log(jnp.sum(jnp.exp(shifted), axis=-1, keepdims=True))
    return scores_col + (shifted - lse)


def _mono_body(root_ref, emb_ref, wih_ref, whh_ref, bih_ref, bhh_ref,
               wout_ref, bout_ref, t_ref, oseq_ref, osc_ref):
    t = t_ref[0]
    D = DECISIONS
    H = HIDDEN
    dn = (((1,), (1,)), ((), ()))
    lane = jax.lax.broadcasted_iota(jnp.int32, (1, K), 1)
    rowi = jax.lax.broadcasted_iota(jnp.int32, (K, 1), 0)
    colD = jax.lax.broadcasted_iota(jnp.int32, (K, D), 1)
    colK = jax.lax.broadcasted_iota(jnp.int32, (K, K), 1)
    colS = jax.lax.broadcasted_iota(jnp.int32, (K, ENV_DEPTH), 1)
    big = jnp.int32(2 ** 31 - 1)
    neg = jnp.float32(-jnp.inf)

    def top64(cand):
        nb = cand.shape[0]
        rowb = jax.lax.broadcasted_iota(jnp.int32, (nb, D), 0)
        colb = jax.lax.broadcasted_iota(jnp.int32, (nb, D), 1)
        rown = jax.lax.broadcasted_iota(jnp.int32, (nb, 1), 0)

        def body(j, carry):
            work, vals_lane, vals_col, a_col, bi_col = carry
            rmax = jnp.max(work, axis=1, keepdims=True)
            rcol = jnp.min(jnp.where(work == rmax, colb, big),
                           axis=1, keepdims=True)
            m = jnp.max(rmax, axis=0, keepdims=True)
            bi = jnp.min(jnp.where(rmax == m, rown, big),
                         axis=0, keepdims=True)
            a = jnp.min(jnp.where(rown == bi, rcol, big),
                        axis=0, keepdims=True)
            sel = rowi == j
            vals_lane = jnp.where(lane == j, m, vals_lane)
            vals_col = jnp.where(sel, m, vals_col)
            a_col = jnp.where(sel, a, a_col)
            bi_col = jnp.where(sel, bi, bi_col)
            work = jnp.where((rowb == bi) & (colb == a), neg, work)
            return work, vals_lane, vals_col, a_col, bi_col

        init = (cand,
                jnp.zeros((1, K), jnp.float32), jnp.zeros((K, 1), jnp.float32),
                jnp.zeros((K, 1), jnp.int32), jnp.zeros((K, 1), jnp.int32))
        _, vals_lane, vals_col, a_col, bi_col = jax.lax.fori_loop(
            0, K, body, init)
        return vals_lane, vals_col, a_col, bi_col

    states = root_ref[...]
    scores_col = jnp.zeros((1, 1), jnp.float32)
    seqs_f = jnp.zeros((K, ENV_DEPTH), jnp.float32)
    vals_lane = None
    for d in range(ENV_DEPTH):
        nb = states.shape[0]
        logits = jnp.dot(states, wout_ref[...],
                         preferred_element_type=jnp.float32) + bout_ref[...]
        cand = _softmax_cand(logits, t, scores_col)
        vals_lane, scores_col, a_col, bi_col = top64(cand)
        hp = jax.lax.Precision.HIGHEST
        onehot_a = (colD == a_col).astype(jnp.float32)
        x = jnp.dot(onehot_a, emb_ref[...], precision=hp,
                    preferred_element_type=jnp.float32)
        if d == 0:
            h = jnp.broadcast_to(states, (K, H))
            seqs_f = jnp.where(colS == d, a_col.astype(jnp.float32), seqs_f)
        else:
            onehot_b = (colK == bi_col).astype(jnp.float32)
            h = jnp.dot(onehot_b, states, precision=hp,
                        preferred_element_type=jnp.float32)
            seqs_f = jnp.dot(onehot_b, seqs_f, precision=hp,
                             preferred_element_type=jnp.float32)
            seqs_f = jnp.where(colS == d, a_col.astype(jnp.float32), seqs_f)
        gi = jax.lax.dot_general(x, wih_ref[...], dn,
                                 preferred_element_type=jnp.float32) + bih_ref[...]
        gh = jax.lax.dot_general(h, whh_ref[...], dn,
                                 preferred_element_type=jnp.float32) + bhh_ref[...]
        i_r, i_z, i_n = gi[:, 0:H], gi[:, H:2 * H], gi[:, 2 * H:3 * H]
        h_r, h_z, h_n = gh[:, 0:H], gh[:, H:2 * H], gh[:, 2 * H:3 * H]
        r = jax.nn.sigmoid(i_r + h_r)
        z = jax.nn.sigmoid(i_z + h_z)
        n = jnp.tanh(i_n + r * h_n)
        states = (1.0 - z) * n + z * h

    osc_ref[...] = jnp.broadcast_to(vals_lane, (BATCH, K))
    seqs_i = seqs_f.astype(jnp.int32)
    for b in range(BATCH):
        oseq_ref[b] = seqs_i


def kernel(env, root_state, emb, W_ih, W_hh, b_ih, b_hh, W_out, b_out, beams, temp):
    H = root_state.shape[1]
    D = W_out.shape[1]
    tempf = jnp.asarray(temp, jnp.float32).reshape(1)
    out_seqs, out_scores = pl.pallas_call(
        _mono_body,
        in_specs=[
            pl.BlockSpec(memory_space=pltpu.VMEM),
            pl.BlockSpec(memory_space=pltpu.VMEM),
            pl.BlockSpec(memory_space=pltpu.VMEM),
            pl.BlockSpec(memory_space=pltpu.VMEM),
            pl.BlockSpec(memory_space=pltpu.VMEM),
            pl.BlockSpec(memory_space=pltpu.VMEM),
            pl.BlockSpec(memory_space=pltpu.VMEM),
            pl.BlockSpec(memory_space=pltpu.VMEM),
            pl.BlockSpec(memory_space=pltpu.SMEM),
        ],
        out_shape=(
            jax.ShapeDtypeStruct((BATCH, K, ENV_DEPTH), jnp.int32),
            jax.ShapeDtypeStruct((BATCH, K), jnp.float32),
        ),
    )(root_state, emb, W_ih, W_hh,
      b_ih.reshape(1, 3 * H), b_hh.reshape(1, 3 * H),
      W_out, b_out.reshape(1, D), tempf)
    return (out_seqs, out_scores)
